# baseline (speedup 1.0000x reference)
import functools

import jax
import jax.numpy as jnp
from jax import lax
from jax.experimental import pallas as pl
from jax.experimental.pallas import tpu as pltpu
from jax.experimental.pallas import tpu_sc as plsc

SH_DEGREE = 4
GFC = 11 + 3 * SH_DEGREE ** 2
C0 = 0.28209479177387814
VOXEL_SIZE = 128
M = 262144
B = 131072

NC = 2
NS = 16
LANES = 16

BT = B // NS
NJ = BT // 128
MT = M // NS
ROWS_PER_CORE = 30


def _sc_scatter_gather_body(ht_hbm, val_hbm, idx_hbm, idx2_hbm,
                            cx_hbm, cy_hbm, cz_hbm,
                            gath_hbm, cg_hbm, stats_hbm,
                            row_sp, idx_vm, idx_fl, val_vm, out_vm, cg_vm,
                            red_vm, stat_vm, sem):
    c = lax.axis_index("c")
    s = lax.axis_index("s")

    pltpu.sync_copy(idx_hbm.at[s], idx_vm)
    pltpu.sync_copy(idx2_hbm.at[s], idx_fl)

    zeros = jnp.zeros((LANES,), jnp.float32)
    stat_vm[0, :] = zeros
    stat_vm[1, :] = zeros

    idx_half = idx_fl.at[pl.ds(c * (BT // 2), BT // 2)]
    for c3, csrc in enumerate((cx_hbm, cy_hbm, cz_hbm)):
        pltpu.sync_copy(csrc.at[idx_half], cg_vm)
        pltpu.sync_copy(cg_vm, cg_hbm.at[c3, s, c])

    def row_step(t, _):
        r = c * ROWS_PER_CORE + t
        valid = r < GFC

        @pl.when(valid)
        def _load():
            pltpu.sync_copy(ht_hbm.at[r, pl.ds(s * MT, MT)],
                            row_sp.at[pl.ds(s * MT, MT)])

        plsc.subcore_barrier()

        @pl.when(valid)
        def _scatter():
            pltpu.sync_copy(val_hbm.at[r, s], val_vm)
            for j0 in range(0, NJ, 16):
                descs = [
                    pltpu.async_copy(val_vm.at[j], row_sp.at[idx_vm.at[j]],
                                     sem, add=True)
                    for j in range(j0, j0 + 16)
                ]
                for d in descs:
                    d.wait()

        plsc.subcore_barrier()

        @pl.when(valid)
        def _gather():
            pltpu.sync_copy(row_sp.at[idx_fl], out_vm)
            pltpu.sync_copy(out_vm, gath_hbm.at[r, s])

        @pl.when(r < 3)
        def _stats():
            pltpu.sync_copy(row_sp.at[pl.ds(s * MT, MT)], red_vm)

            def rbody(i, carry):
                sv, qv = carry
                v = red_vm[pl.ds(i * LANES, LANES)]
                return sv + v, qv + v * v
            sv, qv = lax.fori_loop(0, MT // LANES, rbody, (zeros, zeros))
            stat_vm[0, :] = stat_vm[0, :] + sv
            stat_vm[1, :] = stat_vm[1, :] + qv

        plsc.subcore_barrier()
        return _

    lax.fori_loop(0, ROWS_PER_CORE, row_step, None)

    @pl.when(c == 0)
    def _emit_stats():
        pltpu.sync_copy(stat_vm.at[0], stats_hbm.at[0, s])
        pltpu.sync_copy(stat_vm.at[1], stats_hbm.at[1, s])


def _sc_scatter_gather(hash_table, val4, idx3, idx2, cx, cy, cz):
    mesh = plsc.VectorSubcoreMesh(core_axis_name="c", subcore_axis_name="s")
    f = pl.kernel(
        _sc_scatter_gather_body,
        out_type=(
            jax.ShapeDtypeStruct((GFC, NS, BT), jnp.float32),
            jax.ShapeDtypeStruct((3, NS, NC, BT // 2), jnp.float32),
            jax.ShapeDtypeStruct((2, NS, LANES), jnp.float32),
        ),
        mesh=mesh,
        scratch_types=[
            pltpu.VMEM_SHARED((M,), jnp.float32),
            pltpu.VMEM((NJ, 128), jnp.int32),
            pltpu.VMEM((BT,), jnp.int32),
            pltpu.VMEM((NJ, 128), jnp.float32),
            pltpu.VMEM((BT,), jnp.float32),
            pltpu.VMEM((BT // 2,), jnp.float32),
            pltpu.VMEM((MT,), jnp.float32),
            pltpu.VMEM((2, LANES), jnp.float32),
            pltpu.SemaphoreType.DMA,
        ],
    )
    return f(hash_table, val4, idx3, idx2, cx, cy, cz)


def _tc_dense_body(gath_ref, cg_ref, part_ref, cvec_ref,
                   means_ref, cov_ref, harm_ref, opac_ref):
    p = part_ref[...]
    s1 = jnp.sum(p[0])
    s2 = jnp.sum(p[1])
    n = jnp.float32(3 * M)
    mean = s1 / n
    var = (s2 - s1 * s1 / n) / (n - 1.0)
    rstd = lax.rsqrt(var)

    cvec = cvec_ref[...]
    c_scale = cvec[0:1]
    c_norm = cvec[1:2]
    b_vc = cvec[2:5]

    g = gath_ref[...]
    cg = cg_ref[...]

    meansf = (g[0:3] - mean) * (rstd * c_norm) + cg * c_scale + b_vc
    means_ref[...] = meansf.T

    q = g[3:7]
    qn = q * lax.rsqrt(jnp.sum(q * q, axis=0, keepdims=True))
    r_, x, y, z = qn[0:1], qn[1:2], qn[2:3], qn[3:4]
    sc = jax.nn.sigmoid(g[7:10]) * c_scale
    s0, sA, sB = sc[0:1], sc[1:2], sc[2:3]

    r00 = 1.0 - 2.0 * (y * y + z * z)
    r01 = 2.0 * (x * y - r_ * z)
    r02 = 2.0 * (x * z + r_ * y)
    r10 = 2.0 * (x * y + r_ * z)
    r11 = 1.0 - 2.0 * (x * x + z * z)
    r12 = 2.0 * (y * z - r_ * x)
    r20 = 2.0 * (x * z - r_ * y)
    r21 = 2.0 * (y * z + r_ * x)
    r22 = 1.0 - 2.0 * (x * x + y * y)

    l00, l01, l02 = r00 * s0, r01 * sA, r02 * sB
    l10, l11, l12 = r10 * s0, r11 * sA, r12 * sB
    l20, l21, l22 = r20 * s0, r21 * sA, r22 * sB

    c00 = l00 * l00 + l01 * l01 + l02 * l02
    c01 = l00 * l10 + l01 * l11 + l02 * l12
    c02 = l00 * l20 + l01 * l21 + l02 * l22
    c11 = l10 * l10 + l11 * l11 + l12 * l12
    c12 = l10 * l20 + l11 * l21 + l12 * l22
    c22 = l20 * l20 + l21 * l21 + l22 * l22
    covf = jnp.concatenate(
        [c00, c01, c02, c01, c11, c12, c02, c12, c22], axis=0)
    cov_ref[...] = covf.T

    opac_ref[...] = jax.nn.sigmoid(g[10:11] - 4.0)

    h_low = (jax.nn.sigmoid(g[11:14]) - 0.5) / C0
    harmf = jnp.concatenate([h_low, g[14:GFC]], axis=0)
    harm_ref[...] = harmf.T


def _tc_dense(gath, cg, partials, cvec):
    NB = 512
    grid = (B // NB,)
    return pl.pallas_call(
        _tc_dense_body,
        grid=grid,
        in_specs=[
            pl.BlockSpec((GFC, NB), lambda i: (0, i)),
            pl.BlockSpec((3, NB), lambda i: (0, i)),
            pl.BlockSpec((2, NS, LANES), lambda i: (0, 0, 0)),
            pl.BlockSpec((8, 1), lambda i: (0, 0)),
        ],
        out_specs=[
            pl.BlockSpec((NB, 3), lambda i: (i, 0)),
            pl.BlockSpec((NB, 9), lambda i: (i, 0)),
            pl.BlockSpec((NB, 48), lambda i: (i, 0)),
            pl.BlockSpec((1, NB), lambda i: (0, i)),
        ],
        out_shape=[
            jax.ShapeDtypeStruct((B, 3), jnp.float32),
            jax.ShapeDtypeStruct((B, 9), jnp.float32),
            jax.ShapeDtypeStruct((B, 48), jnp.float32),
            jax.ShapeDtypeStruct((1, B), jnp.float32),
        ],
    )(gath, cg, partials, cvec)


@jax.jit
def kernel(hash_table, val, camera_center, far, idx, coordinates):
    far_s = far[0]

    val4 = val.reshape(GFC, NS, NJ, 128)
    idx3 = idx.reshape(NS, NJ, 128)
    idx2 = idx.reshape(NS, BT)
    coordsf = coordinates.astype(jnp.float32)
    cx = coordsf[:, 0]
    cy = coordsf[:, 1]
    cz = coordsf[:, 2]

    gath4, cg5, partials = _sc_scatter_gather(hash_table, val4, idx3, idx2,
                                              cx, cy, cz)
    gath = gath4.reshape(GFC, B)
    cg = cg5.reshape(3, B)

    c_scale = 2.0 * far_s / VOXEL_SIZE
    c_norm = c_scale / 6.0
    offset = lax.stop_gradient(
        ((camera_center - far_s) * VOXEL_SIZE / 2.0 / far_s)
        .astype(jnp.int32)).astype(jnp.float32)
    b_vc = offset * c_scale + far_s / VOXEL_SIZE
    cvec = jnp.concatenate(
        [jnp.stack([c_scale, c_norm]), b_vc, jnp.zeros((3,), jnp.float32)]
    ).reshape(8, 1)

    means_g, cov_g, harm_g, opacf = _tc_dense(gath, cg, partials, cvec)

    means = means_g.reshape(1, B, 3)
    cov = cov_g.reshape(1, B, 3, 3)
    harmonics = harm_g.reshape(1, B, 3, SH_DEGREE ** 2)
    opac = opacf.reshape(1, B)
    return means, cov, harmonics, opac


# speedup vs baseline: 1.5347x; 1.5347x over previous
import functools

import jax
import jax.numpy as jnp
from jax import lax
from jax.experimental import pallas as pl
from jax.experimental.pallas import tpu as pltpu
from jax.experimental.pallas import tpu_sc as plsc

SH_DEGREE = 4
GFC = 11 + 3 * SH_DEGREE ** 2
C0 = 0.28209479177387814
VOXEL_SIZE = 128
M = 262144
B = 131072

NC = 2
NS = 16
LANES = 16

BT = B // NS
NJ = BT // 128
MT = M // NS
ROWS_PER_CORE = 30


def _sc_scatter_gather_body(ht_hbm, val_hbm, idx_hbm, idx2_hbm,
                            cx_hbm, cy_hbm, cz_hbm,
                            gath_hbm, cg_hbm, stats_hbm,
                            row_sp, idx_vm, idx_fl, val_vm, out_vm, cg_vm,
                            red_vm, stat_vm, sem):
    c = lax.axis_index("c")
    s = lax.axis_index("s")

    pltpu.sync_copy(idx_hbm.at[s], idx_vm)
    pltpu.sync_copy(idx2_hbm.at[s], idx_fl)

    zeros = jnp.zeros((LANES,), jnp.float32)
    stat_vm[0, :] = zeros
    stat_vm[1, :] = zeros

    idx_half = idx_fl.at[pl.ds(c * (BT // 2), BT // 2)]
    for c3, csrc in enumerate((cx_hbm, cy_hbm, cz_hbm)):
        pltpu.sync_copy(csrc.at[idx_half], cg_vm)
        pltpu.sync_copy(cg_vm, cg_hbm.at[c3, s, c])

    def row_step(t, _):
        r = c * ROWS_PER_CORE + t
        valid = r < GFC

        @pl.when(valid)
        def _load():
            pltpu.sync_copy(ht_hbm.at[r, pl.ds(s * MT, MT)],
                            row_sp.at[pl.ds(s * MT, MT)])

        plsc.subcore_barrier()

        @pl.when(valid)
        def _scatter():
            pltpu.sync_copy(val_hbm.at[r, s], val_vm)
            for j0 in range(0, NJ, 16):
                descs = [
                    pltpu.async_copy(val_vm.at[j], row_sp.at[idx_vm.at[j]],
                                     sem, add=True)
                    for j in range(j0, j0 + 16)
                ]
                for d in descs:
                    d.wait()

        plsc.subcore_barrier()

        @pl.when(valid)
        def _gather():
            pltpu.sync_copy(row_sp.at[idx_fl], out_vm)
            pltpu.sync_copy(out_vm, gath_hbm.at[r, s])

        @pl.when(r < 3)
        def _stats():
            pltpu.sync_copy(row_sp.at[pl.ds(s * MT, MT)], red_vm)

            def rbody(i, carry):
                sv, qv = carry
                v = red_vm[pl.ds(i * LANES, LANES)]
                return sv + v, qv + v * v
            sv, qv = lax.fori_loop(0, MT // LANES, rbody, (zeros, zeros))
            stat_vm[0, :] = stat_vm[0, :] + sv
            stat_vm[1, :] = stat_vm[1, :] + qv

        plsc.subcore_barrier()
        return _

    lax.fori_loop(0, ROWS_PER_CORE, row_step, None)

    @pl.when(c == 0)
    def _emit_stats():
        pltpu.sync_copy(stat_vm.at[0], stats_hbm.at[0, s])
        pltpu.sync_copy(stat_vm.at[1], stats_hbm.at[1, s])


def _sc_scatter_gather(hash_table, val4, idx3, idx2, cx, cy, cz):
    mesh = plsc.VectorSubcoreMesh(core_axis_name="c", subcore_axis_name="s")
    f = pl.kernel(
        _sc_scatter_gather_body,
        out_type=(
            jax.ShapeDtypeStruct((GFC, NS, BT), jnp.float32),
            jax.ShapeDtypeStruct((3, NS, NC, BT // 2), jnp.float32),
            jax.ShapeDtypeStruct((2, NS, LANES), jnp.float32),
        ),
        mesh=mesh,
        scratch_types=[
            pltpu.VMEM_SHARED((M,), jnp.float32),
            pltpu.VMEM((NJ, 128), jnp.int32),
            pltpu.VMEM((BT,), jnp.int32),
            pltpu.VMEM((NJ, 128), jnp.float32),
            pltpu.VMEM((BT,), jnp.float32),
            pltpu.VMEM((BT // 2,), jnp.float32),
            pltpu.VMEM((MT,), jnp.float32),
            pltpu.VMEM((2, LANES), jnp.float32),
            pltpu.SemaphoreType.DMA,
        ],
    )
    return f(hash_table, val4, idx3, idx2, cx, cy, cz)


def _tc_dense_body(gath_ref, cg_ref, part_ref, cvec_ref,
                   means_ref, cov_ref, harm_ref, opac_ref):
    p = part_ref[...]
    s1 = jnp.sum(p[0])
    s2 = jnp.sum(p[1])
    n = jnp.float32(3 * M)
    mean = s1 / n
    var = (s2 - s1 * s1 / n) / (n - 1.0)
    rstd = lax.rsqrt(var)

    cvec = cvec_ref[...]
    c_scale = cvec[0:1]
    c_norm = cvec[1:2]
    b_vc = cvec[2:5]

    g = gath_ref[...]
    cg = cg_ref[...]

    means_ref[...] = (g[0:3] - mean) * (rstd * c_norm) + cg * c_scale + b_vc

    q = g[3:7]
    qn = q * lax.rsqrt(jnp.sum(q * q, axis=0, keepdims=True))
    r_, x, y, z = qn[0:1], qn[1:2], qn[2:3], qn[3:4]
    sc = jax.nn.sigmoid(g[7:10]) * c_scale
    s0, sA, sB = sc[0:1], sc[1:2], sc[2:3]

    r00 = 1.0 - 2.0 * (y * y + z * z)
    r01 = 2.0 * (x * y - r_ * z)
    r02 = 2.0 * (x * z + r_ * y)
    r10 = 2.0 * (x * y + r_ * z)
    r11 = 1.0 - 2.0 * (x * x + z * z)
    r12 = 2.0 * (y * z - r_ * x)
    r20 = 2.0 * (x * z - r_ * y)
    r21 = 2.0 * (y * z + r_ * x)
    r22 = 1.0 - 2.0 * (x * x + y * y)

    l00, l01, l02 = r00 * s0, r01 * sA, r02 * sB
    l10, l11, l12 = r10 * s0, r11 * sA, r12 * sB
    l20, l21, l22 = r20 * s0, r21 * sA, r22 * sB

    c00 = l00 * l00 + l01 * l01 + l02 * l02
    c01 = l00 * l10 + l01 * l11 + l02 * l12
    c02 = l00 * l20 + l01 * l21 + l02 * l22
    c11 = l10 * l10 + l11 * l11 + l12 * l12
    c12 = l10 * l20 + l11 * l21 + l12 * l22
    c22 = l20 * l20 + l21 * l21 + l22 * l22
    cov_ref[...] = jnp.concatenate(
        [c00, c01, c02, c01, c11, c12, c02, c12, c22], axis=0)

    opac_ref[...] = jax.nn.sigmoid(g[10:11] - 4.0)

    h_low = (jax.nn.sigmoid(g[11:14]) - 0.5) / C0
    harm_ref[...] = jnp.concatenate([h_low, g[14:GFC]], axis=0)


def _tc_dense(gath, cg, partials, cvec):
    NB = 2048
    grid = (B // NB,)
    return pl.pallas_call(
        _tc_dense_body,
        grid=grid,
        in_specs=[
            pl.BlockSpec((GFC, NB), lambda i: (0, i)),
            pl.BlockSpec((3, NB), lambda i: (0, i)),
            pl.BlockSpec((2, NS, LANES), lambda i: (0, 0, 0)),
            pl.BlockSpec((8, 1), lambda i: (0, 0)),
        ],
        out_specs=[
            pl.BlockSpec((3, NB), lambda i: (0, i)),
            pl.BlockSpec((9, NB), lambda i: (0, i)),
            pl.BlockSpec((48, NB), lambda i: (0, i)),
            pl.BlockSpec((1, NB), lambda i: (0, i)),
        ],
        out_shape=[
            jax.ShapeDtypeStruct((3, B), jnp.float32),
            jax.ShapeDtypeStruct((9, B), jnp.float32),
            jax.ShapeDtypeStruct((48, B), jnp.float32),
            jax.ShapeDtypeStruct((1, B), jnp.float32),
        ],
    )(gath, cg, partials, cvec)


@jax.jit
def kernel(hash_table, val, camera_center, far, idx, coordinates):
    far_s = far[0]

    val4 = val.reshape(GFC, NS, NJ, 128)
    idx3 = idx.reshape(NS, NJ, 128)
    idx2 = idx.reshape(NS, BT)
    coordsf = coordinates.astype(jnp.float32)
    cx = coordsf[:, 0]
    cy = coordsf[:, 1]
    cz = coordsf[:, 2]

    gath4, cg5, partials = _sc_scatter_gather(hash_table, val4, idx3, idx2,
                                              cx, cy, cz)
    gath = gath4.reshape(GFC, B)
    cg = cg5.reshape(3, B)

    c_scale = 2.0 * far_s / VOXEL_SIZE
    c_norm = c_scale / 6.0
    offset = lax.stop_gradient(
        ((camera_center - far_s) * VOXEL_SIZE / 2.0 / far_s)
        .astype(jnp.int32)).astype(jnp.float32)
    b_vc = offset * c_scale + far_s / VOXEL_SIZE
    cvec = jnp.concatenate(
        [jnp.stack([c_scale, c_norm]), b_vc, jnp.zeros((3,), jnp.float32)]
    ).reshape(8, 1)

    meansf, covf, harmf, opacf = _tc_dense(gath, cg, partials, cvec)

    means = meansf.T.reshape(1, B, 3)
    cov = covf.T.reshape(1, B, 3, 3)
    harmonics = harmf.T.reshape(1, B, 3, SH_DEGREE ** 2)
    opac = opacf.reshape(1, B)
    return means, cov, harmonics, opac


# speedup vs baseline: 1.8549x; 1.2086x over previous
import functools

import jax
import jax.numpy as jnp
from jax import lax
from jax.experimental import pallas as pl
from jax.experimental.pallas import tpu as pltpu
from jax.experimental.pallas import tpu_sc as plsc

SH_DEGREE = 4
GFC = 11 + 3 * SH_DEGREE ** 2
C0 = 0.28209479177387814
VOXEL_SIZE = 128
M = 262144
B = 131072

NC = 2
NS = 16
LANES = 16

BT = B // NS
NJ = BT // 128
MT = M // NS
ROWS_PER_CORE = 30


def _sc_scatter_gather_body(ht_hbm, val_hbm, idx_hbm, idx2_hbm,
                            cx_hbm, cy_hbm, cz_hbm,
                            gath_hbm, cg_hbm, stats_hbm,
                            row_a, row_b, idx_vm, idx_fl, val_a, val_b,
                            out_vm, cg_vm, red_vm, stat_vm, sem, semR,
                            semV):
    c = lax.axis_index("c")
    s = lax.axis_index("s")

    pltpu.sync_copy(idx_hbm.at[s], idx_vm)
    pltpu.sync_copy(idx2_hbm.at[s], idx_fl)

    zeros = jnp.zeros((LANES,), jnp.float32)
    stat_vm[0, :] = zeros
    stat_vm[1, :] = zeros

    idx_half = idx_fl.at[pl.ds(c * (BT // 2), BT // 2)]
    for c3, csrc in enumerate((cx_hbm, cy_hbm, cz_hbm)):
        pltpu.sync_copy(csrc.at[idx_half], cg_vm)
        pltpu.sync_copy(cg_vm, cg_hbm.at[c3, s, c])

    ms = pl.ds(s * MT, MT)

    def row_clamped(tt):
        return jnp.minimum(c * ROWS_PER_CORE + tt, GFC - 1)

    pltpu.async_copy(ht_hbm.at[row_clamped(0), ms], row_a.at[ms], semR)
    pltpu.async_copy(val_hbm.at[row_clamped(0), pl.ds(s * NJ, NJ)], val_a, semV)

    def do_phase1(rowbuf, valbuf, orow, oval, t):
        pltpu.async_copy(ht_hbm.at[row_clamped(t + 1), ms], orow.at[ms],
                         semR)
        pltpu.async_copy(val_hbm.at[row_clamped(t + 1), pl.ds(s * NJ, NJ)],
                         oval, semV)

    def do_scatter(rowbuf, valbuf):
        for j0 in range(0, NJ, 16):
            descs = [
                pltpu.async_copy(valbuf.at[j], rowbuf.at[idx_vm.at[j]],
                                 sem, add=True)
                for j in range(j0, j0 + 16)
            ]
            for d in descs:
                d.wait()

    def do_gather(rowbuf, r):
        pltpu.sync_copy(rowbuf.at[idx_fl], out_vm)
        pltpu.sync_copy(out_vm, gath_hbm.at[r, s])

    def do_stats(rowbuf):
        pltpu.sync_copy(rowbuf.at[ms], red_vm)

        def rbody(i, carry):
            sv, qv = carry
            v = red_vm[pl.ds(i * LANES, LANES)]
            return sv + v, qv + v * v
        sv, qv = lax.fori_loop(0, MT // LANES, rbody, (zeros, zeros))
        stat_vm[0, :] = stat_vm[0, :] + sv
        stat_vm[1, :] = stat_vm[1, :] + qv

    def row_step(t, _):
        r = c * ROWS_PER_CORE + t
        valid = r < GFC
        p0 = lax.rem(t, 2) == 0

        pltpu.make_async_copy(ht_hbm.at[0, ms], row_a.at[ms], semR).wait()
        pltpu.make_async_copy(val_hbm.at[0, pl.ds(0, NJ)], val_a, semV).wait()

        plsc.subcore_barrier()

        @pl.when(p0)
        def _f0():
            do_phase1(row_a, val_a, row_b, val_b, t)

        @pl.when(jnp.logical_not(p0))
        def _f1():
            do_phase1(row_b, val_b, row_a, val_a, t)

        @pl.when(jnp.logical_and(p0, valid))
        def _s0():
            do_scatter(row_a, val_a)

        @pl.when(jnp.logical_and(jnp.logical_not(p0), valid))
        def _s1():
            do_scatter(row_b, val_b)

        plsc.subcore_barrier()

        @pl.when(jnp.logical_and(p0, valid))
        def _g0():
            do_gather(row_a, r)

        @pl.when(jnp.logical_and(jnp.logical_not(p0), valid))
        def _g1():
            do_gather(row_b, r)

        @pl.when(jnp.logical_and(p0, r < 3))
        def _t0():
            do_stats(row_a)

        @pl.when(jnp.logical_and(jnp.logical_not(p0), r < 3))
        def _t1():
            do_stats(row_b)

        plsc.subcore_barrier()
        return _

    lax.fori_loop(0, ROWS_PER_CORE, row_step, None)

    pltpu.make_async_copy(ht_hbm.at[0, ms], row_a.at[ms], semR).wait()
    pltpu.make_async_copy(val_hbm.at[0, pl.ds(0, NJ)], val_a, semV).wait()

    @pl.when(c == 0)
    def _emit_stats():
        pltpu.sync_copy(stat_vm.at[0], stats_hbm.at[0, s])
        pltpu.sync_copy(stat_vm.at[1], stats_hbm.at[1, s])


def _sc_scatter_gather(hash_table, val4, idx3, idx2, cx, cy, cz):
    mesh = plsc.VectorSubcoreMesh(core_axis_name="c", subcore_axis_name="s")
    f = pl.kernel(
        _sc_scatter_gather_body,
        out_type=(
            jax.ShapeDtypeStruct((GFC, NS, BT), jnp.float32),
            jax.ShapeDtypeStruct((3, NS, NC, BT // 2), jnp.float32),
            jax.ShapeDtypeStruct((2, NS, LANES), jnp.float32),
        ),
        mesh=mesh,
        scratch_types=[
            pltpu.VMEM_SHARED((M,), jnp.float32),
            pltpu.VMEM_SHARED((M,), jnp.float32),
            pltpu.VMEM((NJ, 128), jnp.int32),
            pltpu.VMEM((BT,), jnp.int32),
            pltpu.VMEM((NJ, 128), jnp.float32),
            pltpu.VMEM((NJ, 128), jnp.float32),
            pltpu.VMEM((BT,), jnp.float32),
            pltpu.VMEM((BT // 2,), jnp.float32),
            pltpu.VMEM((MT,), jnp.float32),
            pltpu.VMEM((2, LANES), jnp.float32),
            pltpu.SemaphoreType.DMA,
            pltpu.SemaphoreType.DMA,
            pltpu.SemaphoreType.DMA,
        ],
    )
    return f(hash_table, val4, idx3, idx2, cx, cy, cz)


def _tc_dense_body(gath_ref, cg_ref, part_ref, cvec_ref,
                   means_ref, cov_ref, harm_ref, opac_ref):
    p = part_ref[...]
    s1 = jnp.sum(p[0])
    s2 = jnp.sum(p[1])
    n = jnp.float32(3 * M)
    mean = s1 / n
    var = (s2 - s1 * s1 / n) / (n - 1.0)
    rstd = lax.rsqrt(var)

    cvec = cvec_ref[...]
    c_scale = cvec[0:1]
    c_norm = cvec[1:2]
    b_vc = cvec[2:5]

    g = gath_ref[...]
    cg = cg_ref[...]

    means_ref[...] = (g[0:3] - mean) * (rstd * c_norm) + cg * c_scale + b_vc

    q = g[3:7]
    qn = q * lax.rsqrt(jnp.sum(q * q, axis=0, keepdims=True))
    r_, x, y, z = qn[0:1], qn[1:2], qn[2:3], qn[3:4]
    sc = jax.nn.sigmoid(g[7:10]) * c_scale
    s0, sA, sB = sc[0:1], sc[1:2], sc[2:3]

    r00 = 1.0 - 2.0 * (y * y + z * z)
    r01 = 2.0 * (x * y - r_ * z)
    r02 = 2.0 * (x * z + r_ * y)
    r10 = 2.0 * (x * y + r_ * z)
    r11 = 1.0 - 2.0 * (x * x + z * z)
    r12 = 2.0 * (y * z - r_ * x)
    r20 = 2.0 * (x * z - r_ * y)
    r21 = 2.0 * (y * z + r_ * x)
    r22 = 1.0 - 2.0 * (x * x + y * y)

    l00, l01, l02 = r00 * s0, r01 * sA, r02 * sB
    l10, l11, l12 = r10 * s0, r11 * sA, r12 * sB
    l20, l21, l22 = r20 * s0, r21 * sA, r22 * sB

    c00 = l00 * l00 + l01 * l01 + l02 * l02
    c01 = l00 * l10 + l01 * l11 + l02 * l12
    c02 = l00 * l20 + l01 * l21 + l02 * l22
    c11 = l10 * l10 + l11 * l11 + l12 * l12
    c12 = l10 * l20 + l11 * l21 + l12 * l22
    c22 = l20 * l20 + l21 * l21 + l22 * l22
    cov_ref[...] = jnp.concatenate(
        [c00, c01, c02, c01, c11, c12, c02, c12, c22], axis=0)

    opac_ref[...] = jax.nn.sigmoid(g[10:11] - 4.0)

    h_low = (jax.nn.sigmoid(g[11:14]) - 0.5) / C0
    harm_ref[...] = jnp.concatenate([h_low, g[14:GFC]], axis=0)


def _tc_dense(gath, cg, partials, cvec):
    NB = 2048
    grid = (B // NB,)
    return pl.pallas_call(
        _tc_dense_body,
        grid=grid,
        in_specs=[
            pl.BlockSpec((GFC, NB), lambda i: (0, i)),
            pl.BlockSpec((3, NB), lambda i: (0, i)),
            pl.BlockSpec((2, NS, LANES), lambda i: (0, 0, 0)),
            pl.BlockSpec((8, 1), lambda i: (0, 0)),
        ],
        out_specs=[
            pl.BlockSpec((3, NB), lambda i: (0, i)),
            pl.BlockSpec((9, NB), lambda i: (0, i)),
            pl.BlockSpec((48, NB), lambda i: (0, i)),
            pl.BlockSpec((1, NB), lambda i: (0, i)),
        ],
        out_shape=[
            jax.ShapeDtypeStruct((3, B), jnp.float32),
            jax.ShapeDtypeStruct((9, B), jnp.float32),
            jax.ShapeDtypeStruct((48, B), jnp.float32),
            jax.ShapeDtypeStruct((1, B), jnp.float32),
        ],
    )(gath, cg, partials, cvec)


@jax.jit
def kernel(hash_table, val, camera_center, far, idx, coordinates):
    far_s = far[0]

    val4 = val.reshape(GFC, NS * NJ, 128)
    idx3 = idx.reshape(NS, NJ, 128)
    idx2 = idx.reshape(NS, BT)
    coordsf = coordinates.astype(jnp.float32)
    cx = coordsf[:, 0]
    cy = coordsf[:, 1]
    cz = coordsf[:, 2]

    gath4, cg5, partials = _sc_scatter_gather(hash_table, val4, idx3, idx2,
                                              cx, cy, cz)
    gath = gath4.reshape(GFC, B)
    cg = cg5.reshape(3, B)

    c_scale = 2.0 * far_s / VOXEL_SIZE
    c_norm = c_scale / 6.0
    offset = lax.stop_gradient(
        ((camera_center - far_s) * VOXEL_SIZE / 2.0 / far_s)
        .astype(jnp.int32)).astype(jnp.float32)
    b_vc = offset * c_scale + far_s / VOXEL_SIZE
    cvec = jnp.concatenate(
        [jnp.stack([c_scale, c_norm]), b_vc, jnp.zeros((3,), jnp.float32)]
    ).reshape(8, 1)

    meansf, covf, harmf, opacf = _tc_dense(gath, cg, partials, cvec)

    means = meansf.T.reshape(1, B, 3)
    cov = covf.T.reshape(1, B, 3, 3)
    harmonics = harmf.T.reshape(1, B, 3, SH_DEGREE ** 2)
    opac = opacf.reshape(1, B)
    return means, cov, harmonics, opac


# speedup vs baseline: 1.8828x; 1.0150x over previous
import jax
import jax.numpy as jnp
from jax import lax
from jax.experimental import pallas as pl
from jax.experimental.pallas import tpu as pltpu
from jax.experimental.pallas import tpu_sc as plsc

SH_DEGREE = 4
GFC = 11 + 3 * SH_DEGREE ** 2
C0 = 0.28209479177387814
VOXEL_SIZE = 128
M = 262144
B = 131072

NC = 2
NS = 16
LANES = 16

BT = B // NS
NJ = BT // 128
MT = M // NS
ROWS_PER_CORE = 30


def _sc_scatter_gather_body(ht_hbm, val_hbm, idx_hbm, idx2_hbm,
                            cx_hbm, cy_hbm, cz_hbm,
                            gath_hbm, cg_hbm, stats_hbm,
                            row_a, row_b, idx_vm, idx_fl, val_a, val_b,
                            out_vm, cg_vm, red_vm, stat_vm, sem, semR,
                            semV):
    c = lax.axis_index("c")
    s = lax.axis_index("s")

    pltpu.sync_copy(idx_hbm.at[s], idx_vm)
    pltpu.sync_copy(idx2_hbm.at[s], idx_fl)

    zeros = jnp.zeros((LANES,), jnp.float32)
    stat_vm[0, :] = zeros
    stat_vm[1, :] = zeros

    ms = pl.ds(s * MT, MT)

    def row_clamped(tt):
        return jnp.minimum(c * ROWS_PER_CORE + tt, GFC - 1)

    pltpu.async_copy(ht_hbm.at[row_clamped(0), ms], row_a.at[ms], semR)
    pltpu.async_copy(val_hbm.at[row_clamped(0), pl.ds(s * NJ, NJ)], val_a, semV)

    idx_half = idx_fl.at[pl.ds(c * (BT // 2), BT // 2)]
    for c3, csrc in enumerate((cx_hbm, cy_hbm, cz_hbm)):
        pltpu.sync_copy(csrc.at[idx_half], cg_vm)
        pltpu.sync_copy(cg_vm, cg_hbm.at[c3, s, c])

    def do_phase1(rowbuf, valbuf, orow, oval, t):
        pltpu.async_copy(ht_hbm.at[row_clamped(t + 1), ms], orow.at[ms],
                         semR)
        pltpu.async_copy(val_hbm.at[row_clamped(t + 1), pl.ds(s * NJ, NJ)],
                         oval, semV)

    def do_scatter(rowbuf, valbuf):
        window = []
        for j in range(NJ):
            window.append(pltpu.async_copy(
                valbuf.at[j], rowbuf.at[idx_vm.at[j]], sem, add=True))
            if len(window) >= 16:
                window.pop(0).wait()
        for d in window:
            d.wait()

    def do_gather(rowbuf, r):
        pltpu.sync_copy(rowbuf.at[idx_fl], out_vm)
        pltpu.sync_copy(out_vm, gath_hbm.at[r, s])

    def do_stats(rowbuf):
        pltpu.sync_copy(rowbuf.at[ms], red_vm)

        def rbody(i, carry):
            sv, qv = carry
            v = red_vm[pl.ds(i * LANES, LANES)]
            return sv + v, qv + v * v
        sv, qv = lax.fori_loop(0, MT // LANES, rbody, (zeros, zeros))
        stat_vm[0, :] = stat_vm[0, :] + sv
        stat_vm[1, :] = stat_vm[1, :] + qv

    def row_step(t, _):
        r = c * ROWS_PER_CORE + t
        valid = r < GFC
        p0 = lax.rem(t, 2) == 0

        pltpu.make_async_copy(ht_hbm.at[0, ms], row_a.at[ms], semR).wait()
        pltpu.make_async_copy(val_hbm.at[0, pl.ds(0, NJ)], val_a, semV).wait()

        plsc.subcore_barrier()

        @pl.when(p0)
        def _f0():
            do_phase1(row_a, val_a, row_b, val_b, t)

        @pl.when(jnp.logical_not(p0))
        def _f1():
            do_phase1(row_b, val_b, row_a, val_a, t)

        @pl.when(jnp.logical_and(p0, valid))
        def _s0():
            do_scatter(row_a, val_a)

        @pl.when(jnp.logical_and(jnp.logical_not(p0), valid))
        def _s1():
            do_scatter(row_b, val_b)

        plsc.subcore_barrier()

        @pl.when(jnp.logical_and(p0, valid))
        def _g0():
            do_gather(row_a, r)

        @pl.when(jnp.logical_and(jnp.logical_not(p0), valid))
        def _g1():
            do_gather(row_b, r)

        @pl.when(jnp.logical_and(p0, r < 3))
        def _t0():
            do_stats(row_a)

        @pl.when(jnp.logical_and(jnp.logical_not(p0), r < 3))
        def _t1():
            do_stats(row_b)

        plsc.subcore_barrier()
        return _

    lax.fori_loop(0, ROWS_PER_CORE, row_step, None)

    pltpu.make_async_copy(ht_hbm.at[0, ms], row_a.at[ms], semR).wait()
    pltpu.make_async_copy(val_hbm.at[0, pl.ds(0, NJ)], val_a, semV).wait()

    @pl.when(c == 0)
    def _emit_stats():
        pltpu.sync_copy(stat_vm.at[0], stats_hbm.at[0, s])
        pltpu.sync_copy(stat_vm.at[1], stats_hbm.at[1, s])


def _sc_scatter_gather(hash_table, val4, idx3, idx2, cx, cy, cz):
    mesh = plsc.VectorSubcoreMesh(core_axis_name="c", subcore_axis_name="s")
    f = pl.kernel(
        _sc_scatter_gather_body,
        out_type=(
            jax.ShapeDtypeStruct((GFC, NS, BT), jnp.float32),
            jax.ShapeDtypeStruct((3, NS, NC, BT // 2), jnp.float32),
            jax.ShapeDtypeStruct((2, NS, LANES), jnp.float32),
        ),
        mesh=mesh,
        scratch_types=[
            pltpu.VMEM_SHARED((M,), jnp.float32),
            pltpu.VMEM_SHARED((M,), jnp.float32),
            pltpu.VMEM((NJ, 128), jnp.int32),
            pltpu.VMEM((BT,), jnp.int32),
            pltpu.VMEM((NJ, 128), jnp.float32),
            pltpu.VMEM((NJ, 128), jnp.float32),
            pltpu.VMEM((BT,), jnp.float32),
            pltpu.VMEM((BT // 2,), jnp.float32),
            pltpu.VMEM((MT,), jnp.float32),
            pltpu.VMEM((2, LANES), jnp.float32),
            pltpu.SemaphoreType.DMA,
            pltpu.SemaphoreType.DMA,
            pltpu.SemaphoreType.DMA,
        ],
    )
    return f(hash_table, val4, idx3, idx2, cx, cy, cz)


def _tc_dense_body(gath_ref, cg_ref, part_ref, cvec_ref,
                   means_ref, cov_ref, harm_ref, opac_ref):
    p = part_ref[...]
    s1 = jnp.sum(p[0])
    s2 = jnp.sum(p[1])
    n = jnp.float32(3 * M)
    mean = s1 / n
    var = (s2 - s1 * s1 / n) / (n - 1.0)
    rstd = lax.rsqrt(var)

    cvec = cvec_ref[...]
    c_scale = cvec[0:1]
    c_norm = cvec[1:2]
    b_vc = cvec[2:5]

    g = gath_ref[...]
    cg = cg_ref[...]

    means_ref[...] = (g[0:3] - mean) * (rstd * c_norm) + cg * c_scale + b_vc

    q = g[3:7]
    qn = q * lax.rsqrt(jnp.sum(q * q, axis=0, keepdims=True))
    r_, x, y, z = qn[0:1], qn[1:2], qn[2:3], qn[3:4]
    sc = jax.nn.sigmoid(g[7:10]) * c_scale
    s0, sA, sB = sc[0:1], sc[1:2], sc[2:3]

    r00 = 1.0 - 2.0 * (y * y + z * z)
    r01 = 2.0 * (x * y - r_ * z)
    r02 = 2.0 * (x * z + r_ * y)
    r10 = 2.0 * (x * y + r_ * z)
    r11 = 1.0 - 2.0 * (x * x + z * z)
    r12 = 2.0 * (y * z - r_ * x)
    r20 = 2.0 * (x * z - r_ * y)
    r21 = 2.0 * (y * z + r_ * x)
    r22 = 1.0 - 2.0 * (x * x + y * y)

    l00, l01, l02 = r00 * s0, r01 * sA, r02 * sB
    l10, l11, l12 = r10 * s0, r11 * sA, r12 * sB
    l20, l21, l22 = r20 * s0, r21 * sA, r22 * sB

    c00 = l00 * l00 + l01 * l01 + l02 * l02
    c01 = l00 * l10 + l01 * l11 + l02 * l12
    c02 = l00 * l20 + l01 * l21 + l02 * l22
    c11 = l10 * l10 + l11 * l11 + l12 * l12
    c12 = l10 * l20 + l11 * l21 + l12 * l22
    c22 = l20 * l20 + l21 * l21 + l22 * l22
    cov_ref[...] = jnp.concatenate(
        [c00, c01, c02, c01, c11, c12, c02, c12, c22], axis=0)

    opac_ref[...] = jax.nn.sigmoid(g[10:11] - 4.0)

    h_low = (jax.nn.sigmoid(g[11:14]) - 0.5) / C0
    harm_ref[...] = jnp.concatenate([h_low, g[14:GFC]], axis=0)


def _tc_dense(gath, cg, partials, cvec):
    NB = 2048
    grid = (B // NB,)
    return pl.pallas_call(
        _tc_dense_body,
        grid=grid,
        in_specs=[
            pl.BlockSpec((GFC, NB), lambda i: (0, i)),
            pl.BlockSpec((3, NB), lambda i: (0, i)),
            pl.BlockSpec((2, NS, LANES), lambda i: (0, 0, 0)),
            pl.BlockSpec((8, 1), lambda i: (0, 0)),
        ],
        out_specs=[
            pl.BlockSpec((3, NB), lambda i: (0, i)),
            pl.BlockSpec((9, NB), lambda i: (0, i)),
            pl.BlockSpec((48, NB), lambda i: (0, i)),
            pl.BlockSpec((1, NB), lambda i: (0, i)),
        ],
        out_shape=[
            jax.ShapeDtypeStruct((3, B), jnp.float32),
            jax.ShapeDtypeStruct((9, B), jnp.float32),
            jax.ShapeDtypeStruct((48, B), jnp.float32),
            jax.ShapeDtypeStruct((1, B), jnp.float32),
        ],
    )(gath, cg, partials, cvec)


@jax.jit
def kernel(hash_table, val, camera_center, far, idx, coordinates):
    far_s = far[0]

    val4 = val.reshape(GFC, NS * NJ, 128)
    idx3 = idx.reshape(NS, NJ, 128)
    idx2 = idx.reshape(NS, BT)
    coordsf = coordinates.astype(jnp.float32)
    cx = coordsf[:, 0]
    cy = coordsf[:, 1]
    cz = coordsf[:, 2]

    gath4, cg5, partials = _sc_scatter_gather(hash_table, val4, idx3, idx2,
                                              cx, cy, cz)
    gath = gath4.reshape(GFC, B)
    cg = cg5.reshape(3, B)

    c_scale = 2.0 * far_s / VOXEL_SIZE
    c_norm = c_scale / 6.0
    offset = lax.stop_gradient(
        ((camera_center - far_s) * VOXEL_SIZE / 2.0 / far_s)
        .astype(jnp.int32)).astype(jnp.float32)
    b_vc = offset * c_scale + far_s / VOXEL_SIZE
    cvec = jnp.concatenate(
        [jnp.stack([c_scale, c_norm]), b_vc, jnp.zeros((3,), jnp.float32)]
    ).reshape(8, 1)

    meansf, covf, harmf, opacf = _tc_dense(gath, cg, partials, cvec)

    means = meansf.T.reshape(1, B, 3)
    cov = covf.T.reshape(1, B, 3, 3)
    harmonics = harmf.T.reshape(1, B, 3, SH_DEGREE ** 2)
    opac = opacf.reshape(1, B)
    return means, cov, harmonics, opac


# speedup vs baseline: 1.9646x; 1.0435x over previous
import jax
import jax.numpy as jnp
from jax import lax
from jax.experimental import pallas as pl
from jax.experimental.pallas import tpu as pltpu
from jax.experimental.pallas import tpu_sc as plsc

SH_DEGREE = 4
GFC = 11 + 3 * SH_DEGREE ** 2
C0 = 0.28209479177387814
VOXEL_SIZE = 128
M = 262144
B = 131072

NC = 2
NS = 16
LANES = 16

BT = B // NS
NJ = BT // 128
MT = M // NS
ROWS_PER_CORE = 30


def _sc_scatter_gather_body(ht_hbm, val_hbm, idx_hbm, idx2_hbm,
                            cx_hbm, cy_hbm, cz_hbm,
                            gath_hbm, cg_hbm, stats_hbm,
                            row_a, row_b, idx_vm, idx_fl, val_a, val_b,
                            out_vm, cg_vm, red_vm, stat_vm, sem, semR,
                            semV):
    c = lax.axis_index("c")
    s = lax.axis_index("s")

    pltpu.sync_copy(idx_hbm.at[s], idx_vm)
    pltpu.sync_copy(idx2_hbm.at[s], idx_fl)

    zeros = jnp.zeros((LANES,), jnp.float32)
    stat_vm[0, :] = zeros
    stat_vm[1, :] = zeros

    ms = pl.ds(s * MT, MT)

    def row_clamped(tt):
        return jnp.minimum(c * ROWS_PER_CORE + tt, GFC - 1)

    pltpu.async_copy(ht_hbm.at[row_clamped(0), ms], row_a.at[ms], semR)
    pltpu.async_copy(val_hbm.at[row_clamped(0), pl.ds(s * NJ, NJ)], val_a, semV)

    idx_half = idx_fl.at[pl.ds(c * (BT // 2), BT // 2)]
    for c3, csrc in enumerate((cx_hbm, cy_hbm, cz_hbm)):
        pltpu.sync_copy(csrc.at[idx_half], cg_vm)
        pltpu.sync_copy(cg_vm, cg_hbm.at[c3, s, c])

    def do_phase1(rowbuf, valbuf, orow, oval, t):
        pltpu.async_copy(ht_hbm.at[row_clamped(t + 1), ms], orow.at[ms],
                         semR)
        pltpu.async_copy(val_hbm.at[row_clamped(t + 1), pl.ds(s * NJ, NJ)],
                         oval, semV)

    def do_scatter(rowbuf, valbuf):
        window = []
        for j in range(NJ):
            window.append(pltpu.async_copy(
                valbuf.at[j], rowbuf.at[idx_vm.at[j]], sem, add=True))
            if len(window) >= 16:
                window.pop(0).wait()
        for d in window:
            d.wait()

    def do_gather(rowbuf, r):
        pltpu.sync_copy(rowbuf.at[idx_fl], out_vm)
        pltpu.sync_copy(out_vm, gath_hbm.at[r, s])

    def do_stats(rowbuf):
        pltpu.sync_copy(rowbuf.at[ms], red_vm)

        def rbody(i, carry):
            sv, qv = carry
            v = red_vm[pl.ds(i * LANES, LANES)]
            return sv + v, qv + v * v
        sv, qv = lax.fori_loop(0, MT // LANES, rbody, (zeros, zeros))
        stat_vm[0, :] = stat_vm[0, :] + sv
        stat_vm[1, :] = stat_vm[1, :] + qv

    def row_step(t, _):
        r = c * ROWS_PER_CORE + t
        valid = r < GFC
        p0 = lax.rem(t, 2) == 0

        pltpu.make_async_copy(ht_hbm.at[0, ms], row_a.at[ms], semR).wait()
        pltpu.make_async_copy(val_hbm.at[0, pl.ds(0, NJ)], val_a, semV).wait()

        plsc.subcore_barrier()

        @pl.when(p0)
        def _f0():
            do_phase1(row_a, val_a, row_b, val_b, t)

        @pl.when(jnp.logical_not(p0))
        def _f1():
            do_phase1(row_b, val_b, row_a, val_a, t)

        @pl.when(jnp.logical_and(p0, valid))
        def _s0():
            do_scatter(row_a, val_a)

        @pl.when(jnp.logical_and(jnp.logical_not(p0), valid))
        def _s1():
            do_scatter(row_b, val_b)

        plsc.subcore_barrier()

        @pl.when(jnp.logical_and(p0, valid))
        def _g0():
            do_gather(row_a, r)

        @pl.when(jnp.logical_and(jnp.logical_not(p0), valid))
        def _g1():
            do_gather(row_b, r)

        @pl.when(jnp.logical_and(p0, r < 3))
        def _t0():
            do_stats(row_a)

        @pl.when(jnp.logical_and(jnp.logical_not(p0), r < 3))
        def _t1():
            do_stats(row_b)

        plsc.subcore_barrier()
        return _

    lax.fori_loop(0, ROWS_PER_CORE, row_step, None)

    pltpu.make_async_copy(ht_hbm.at[0, ms], row_a.at[ms], semR).wait()
    pltpu.make_async_copy(val_hbm.at[0, pl.ds(0, NJ)], val_a, semV).wait()

    @pl.when(c == 0)
    def _emit_stats():
        pltpu.sync_copy(stat_vm.at[0], stats_hbm.at[0, s])
        pltpu.sync_copy(stat_vm.at[1], stats_hbm.at[1, s])


def _sc_scatter_gather(hash_table, val4, idx3, idx2, cx, cy, cz):
    mesh = plsc.VectorSubcoreMesh(core_axis_name="c", subcore_axis_name="s")
    f = pl.kernel(
        _sc_scatter_gather_body,
        out_type=(
            jax.ShapeDtypeStruct((GFC, NS, BT), jnp.float32),
            jax.ShapeDtypeStruct((3, NS, NC, BT // 2), jnp.float32),
            jax.ShapeDtypeStruct((2, NS, LANES), jnp.float32),
        ),
        mesh=mesh,
        scratch_types=[
            pltpu.VMEM_SHARED((M,), jnp.float32),
            pltpu.VMEM_SHARED((M,), jnp.float32),
            pltpu.VMEM((NJ, 128), jnp.int32),
            pltpu.VMEM((BT,), jnp.int32),
            pltpu.VMEM((NJ, 128), jnp.float32),
            pltpu.VMEM((NJ, 128), jnp.float32),
            pltpu.VMEM((BT,), jnp.float32),
            pltpu.VMEM((BT // 2,), jnp.float32),
            pltpu.VMEM((MT,), jnp.float32),
            pltpu.VMEM((2, LANES), jnp.float32),
            pltpu.SemaphoreType.DMA,
            pltpu.SemaphoreType.DMA,
            pltpu.SemaphoreType.DMA,
        ],
    )
    return f(hash_table, val4, idx3, idx2, cx, cy, cz)


def _tc_dense_body(gath_ref, cg_ref, part_ref, cvec_ref,
                   means_ref, cov_ref, harm_ref, opac_ref):
    p = part_ref[...]
    s1 = jnp.sum(p[0])
    s2 = jnp.sum(p[1])
    n = jnp.float32(3 * M)
    mean = s1 / n
    var = (s2 - s1 * s1 / n) / (n - 1.0)
    rstd = lax.rsqrt(var)

    cvec = cvec_ref[...]
    c_scale = cvec[0:1]
    c_norm = cvec[1:2]
    b_vc = cvec[2:5]

    g = gath_ref[...]
    cg = cg_ref[...]

    means_ref[...] = (g[0:3] - mean) * (rstd * c_norm) + cg * c_scale + b_vc

    q = g[3:7]
    qn = q * lax.rsqrt(jnp.sum(q * q, axis=0, keepdims=True))
    r_, x, y, z = qn[0:1], qn[1:2], qn[2:3], qn[3:4]
    sc = jax.nn.sigmoid(g[7:10]) * c_scale
    s0, sA, sB = sc[0:1], sc[1:2], sc[2:3]

    r00 = 1.0 - 2.0 * (y * y + z * z)
    r01 = 2.0 * (x * y - r_ * z)
    r02 = 2.0 * (x * z + r_ * y)
    r10 = 2.0 * (x * y + r_ * z)
    r11 = 1.0 - 2.0 * (x * x + z * z)
    r12 = 2.0 * (y * z - r_ * x)
    r20 = 2.0 * (x * z - r_ * y)
    r21 = 2.0 * (y * z + r_ * x)
    r22 = 1.0 - 2.0 * (x * x + y * y)

    l00, l01, l02 = r00 * s0, r01 * sA, r02 * sB
    l10, l11, l12 = r10 * s0, r11 * sA, r12 * sB
    l20, l21, l22 = r20 * s0, r21 * sA, r22 * sB

    c00 = l00 * l00 + l01 * l01 + l02 * l02
    c01 = l00 * l10 + l01 * l11 + l02 * l12
    c02 = l00 * l20 + l01 * l21 + l02 * l22
    c11 = l10 * l10 + l11 * l11 + l12 * l12
    c12 = l10 * l20 + l11 * l21 + l12 * l22
    c22 = l20 * l20 + l21 * l21 + l22 * l22
    cov_ref[...] = jnp.concatenate(
        [c00, c01, c02, c01, c11, c12, c02, c12, c22], axis=0)

    opac_ref[...] = jax.nn.sigmoid(g[10:11] - 4.0)

    h_low = (jax.nn.sigmoid(g[11:14]) - 0.5) / C0
    harm_ref[...] = jnp.concatenate([h_low, g[14:GFC]], axis=0)


def _tc_dense(gath, cg, partials, cvec):
    NB = 4096
    grid = (B // NB,)
    return pl.pallas_call(
        _tc_dense_body,
        grid=grid,
        in_specs=[
            pl.BlockSpec((GFC, NB), lambda i: (0, i)),
            pl.BlockSpec((3, NB), lambda i: (0, i)),
            pl.BlockSpec((2, NS, LANES), lambda i: (0, 0, 0)),
            pl.BlockSpec((8, 1), lambda i: (0, 0)),
        ],
        out_specs=[
            pl.BlockSpec((3, NB), lambda i: (0, i)),
            pl.BlockSpec((9, NB), lambda i: (0, i)),
            pl.BlockSpec((48, NB), lambda i: (0, i)),
            pl.BlockSpec((1, NB), lambda i: (0, i)),
        ],
        out_shape=[
            jax.ShapeDtypeStruct((3, B), jnp.float32),
            jax.ShapeDtypeStruct((9, B), jnp.float32),
            jax.ShapeDtypeStruct((48, B), jnp.float32),
            jax.ShapeDtypeStruct((1, B), jnp.float32),
        ],
    )(gath, cg, partials, cvec)


@jax.jit
def kernel(hash_table, val, camera_center, far, idx, coordinates):
    far_s = far[0]

    val4 = val.reshape(GFC, NS * NJ, 128)
    idx3 = idx.reshape(NS, NJ, 128)
    idx2 = idx.reshape(NS, BT)
    coordsf = coordinates.astype(jnp.float32)
    cx = coordsf[:, 0]
    cy = coordsf[:, 1]
    cz = coordsf[:, 2]

    gath4, cg5, partials = _sc_scatter_gather(hash_table, val4, idx3, idx2,
                                              cx, cy, cz)
    gath = gath4.reshape(GFC, B)
    cg = cg5.reshape(3, B)

    c_scale = 2.0 * far_s / VOXEL_SIZE
    c_norm = c_scale / 6.0
    offset = lax.stop_gradient(
        ((camera_center - far_s) * VOXEL_SIZE / 2.0 / far_s)
        .astype(jnp.int32)).astype(jnp.float32)
    b_vc = offset * c_scale + far_s / VOXEL_SIZE
    cvec = jnp.concatenate(
        [jnp.stack([c_scale, c_norm]), b_vc, jnp.zeros((3,), jnp.float32)]
    ).reshape(8, 1)

    meansf, covf, harmf, opacf = _tc_dense(gath, cg, partials, cvec)

    means = meansf.T.reshape(1, B, 3)
    cov = covf.T.reshape(1, B, 3, 3)
    harmonics = harmf.T.reshape(1, B, 3, SH_DEGREE ** 2)
    opac = opacf.reshape(1, B)
    return means, cov, harmonics, opac
